# baseline (speedup 1.0000x reference)
import functools
import jax
import jax.numpy as jnp
from jax import lax
from jax.experimental import pallas as pl
from jax.experimental.pallas import tpu as pltpu
from jax.experimental.pallas import tpu_sc as plsc

N_BINS = 20
NSEG = N_BINS - 1
_NW = 32
_BS = 16
_PER_S = 4 * 512 * 512
_NS_SC = 8
_TOT = _NS_SC * _PER_S
_PER_W = _TOT // _NW
_CHUNK = 16384
_NCHUNK = _PER_W // _CHUNK
_GRP = _CHUNK // 16


def _sc_body(x_hbm, rany_hbm, out_hbm, yrow_v, ytab_v, atab_v, btab_v,
             inb0_v, inb1_v, inb2_v, outb0_v, outb1_v, outb2_v, sem_in, sem_out):
    inb = (inb0_v, inb1_v, inb2_v)
    outb = (outb0_v, outb1_v, outb2_v)
    cid = lax.axis_index("c")
    sid = lax.axis_index("s")
    wid = sid * 2 + cid
    sample = (wid * _NS_SC) // _NW
    base = wid * _PER_W

    pltpu.sync_copy(rany_hbm.at[sample], yrow_v)
    v0 = yrow_v[pl.ds(0, 16)]
    v1 = yrow_v[pl.ds(16, 16)]
    lane = lax.iota(jnp.int32, 16)
    m1 = lane < (N_BINS - 16)
    big = jnp.float32(3.4e38)
    vmin = jnp.minimum(v0, jnp.where(m1, v1, big))
    vmax = jnp.maximum(v0, jnp.where(m1, v1, -big))
    for sh in (8, 4, 2, 1):
        ridx = (lane + sh) & 15
        yrow_v[pl.ds(0, 16)] = vmin
        vmin = jnp.minimum(vmin, plsc.load_gather(yrow_v, [ridx]))
        yrow_v[pl.ds(0, 16)] = vmax
        vmax = jnp.maximum(vmax, plsc.load_gather(yrow_v, [ridx]))
    ymin = vmin
    sc = 1.0 / (vmax - vmin + 1e-5)
    yn0 = (v0 - ymin) * sc
    yn1 = (v1 - ymin) * sc
    ytab_v[pl.ds(0, 16)] = yn0
    ytab_v[pl.ds(16, 16)] = yn1
    dy0 = ytab_v[pl.ds(1, 16)] - yn0
    dy1 = ytab_v[pl.ds(17, 16)] - ytab_v[pl.ds(16, 16)]
    k0 = lane.astype(jnp.float32)
    k1 = k0 + 16.0
    atab_v[pl.ds(0, 16)] = yn0 - k0 * dy0
    atab_v[pl.ds(16, 16)] = yn1 - k1 * dy1
    btab_v[pl.ds(0, 16)] = dy0
    btab_v[pl.ds(16, 16)] = dy1

    nineteen = jnp.float32(NSEG)

    def compute(buf_in, buf_out):
        @plsc.parallel_loop(0, _CHUNK, 16, unroll=16)
        def body(i):
            xv = buf_in[pl.ds(i, 16)]
            t = xv * nineteen
            idx = t.astype(jnp.int32)
            a = plsc.load_gather(atab_v, [idx])
            b = plsc.load_gather(btab_v, [idx])
            buf_out[pl.ds(i, 16)] = a + b * t

    _NB = 3
    cps_in = {}
    for g in range(min(_NB - 1, _NCHUNK)):
        cps_in[g] = pltpu.async_copy(
            x_hbm.at[pl.ds(base + g * _CHUNK, _CHUNK)], inb[g % _NB], sem_in)
    for g in range(_NCHUNK):
        if g + _NB - 1 < _NCHUNK:
            gg = g + _NB - 1
            cps_in[gg] = pltpu.async_copy(
                x_hbm.at[pl.ds(base + gg * _CHUNK, _CHUNK)], inb[gg % _NB], sem_in)
        cps_in[g].wait()
        if g >= _NB:
            pltpu.make_async_copy(
                outb[g % _NB],
                out_hbm.at[pl.ds(base + (g - _NB) * _CHUNK, _CHUNK)],
                sem_out).wait()
        compute(inb[g % _NB], outb[g % _NB])
        pltpu.async_copy(
            outb[g % _NB], out_hbm.at[pl.ds(base + g * _CHUNK, _CHUNK)], sem_out)
    for g in range(max(0, _NCHUNK - _NB), _NCHUNK):
        pltpu.make_async_copy(
            outb[g % _NB], out_hbm.at[pl.ds(base + g * _CHUNK, _CHUNK)],
            sem_out).wait()


def _sc_lut(x_flat, ran_y_pad):
    mesh = plsc.VectorSubcoreMesh(core_axis_name="c", subcore_axis_name="s")
    k = functools.partial(
        pl.kernel,
        out_type=jax.ShapeDtypeStruct((_TOT,), jnp.float32),
        mesh=mesh,
        compiler_params=pltpu.CompilerParams(needs_layout_passes=False),
        scratch_types=[
            pltpu.VMEM((32,), jnp.float32),
            pltpu.VMEM((48,), jnp.float32),
            pltpu.VMEM((48,), jnp.float32),
            pltpu.VMEM((48,), jnp.float32),
            pltpu.VMEM((_CHUNK,), jnp.float32),
            pltpu.VMEM((_CHUNK,), jnp.float32),
            pltpu.VMEM((_CHUNK,), jnp.float32),
            pltpu.VMEM((_CHUNK,), jnp.float32),
            pltpu.VMEM((_CHUNK,), jnp.float32),
            pltpu.VMEM((_CHUNK,), jnp.float32),
            pltpu.SemaphoreType.DMA,
            pltpu.SemaphoreType.DMA,
        ],
    )(_sc_body)
    return k(x_flat, ran_y_pad)


_ROWS = 1024
_COLS = 1024
_BLK_R = 512


def _lut_tc_kernel(y_ref, x_ref, o_ref):
    y = y_ref[0, 0, :]
    ymin = jnp.min(y)
    ymax = jnp.max(y)
    yn = (y - ymin) / (ymax - ymin + 1e-5)
    y0 = yn[:NSEG]
    dy = yn[1:] - yn[:NSEG]
    hb = lax.bitcast_convert_type(y0.astype(jnp.bfloat16), jnp.uint16)
    lb = lax.bitcast_convert_type(dy.astype(jnp.bfloat16), jnp.uint16)
    tab = (hb.astype(jnp.int32) << 16) | lb.astype(jnp.int32)
    x = x_ref[0]
    t = x * jnp.float32(NSEG)
    idxf = jnp.clip(jnp.floor(t), 0.0, NSEG - 1)
    idx = idxf.astype(jnp.int32)
    frac = t - idxf
    tabb = jnp.broadcast_to(tab.reshape(1, NSEG), (_BLK_R, NSEG))
    g = jnp.take_along_axis(tabb, idx, axis=1)
    y0v = lax.bitcast_convert_type(g & jnp.int32(-65536), jnp.float32)
    dyv = lax.bitcast_convert_type(g << 16, jnp.float32)
    o_ref[0] = y0v + dyv * frac


def _tc_lut(x2, y3):
    nb = _BS - _NS_SC
    return pl.pallas_call(
        _lut_tc_kernel,
        grid=(nb, _ROWS // _BLK_R),
        in_specs=[
            pl.BlockSpec((1, 1, N_BINS), lambda i, j: (i + _NS_SC, 0, 0)),
            pl.BlockSpec((1, _BLK_R, _COLS), lambda i, j: (i + _NS_SC, j, 0)),
        ],
        out_specs=pl.BlockSpec((1, _BLK_R, _COLS), lambda i, j: (i, j, 0)),
        out_shape=jax.ShapeDtypeStruct((nb, _ROWS, _COLS), jnp.float32),
        compiler_params=pltpu.CompilerParams(
            dimension_semantics=("parallel", "parallel"),
        ),
    )(y3, x2)


def kernel(x, ran_y):
    sz = x.shape
    x_flat = x.reshape(-1)
    ran_y_pad = jnp.pad(ran_y, ((0, 0), (0, 32 - N_BINS)))
    sc_out = _sc_lut(x_flat, ran_y_pad)
    x2 = x.reshape(_BS, _ROWS, _COLS)
    y3 = ran_y.reshape(_BS, 1, N_BINS)
    tc_out = _tc_lut(x2, y3)
    out = jnp.concatenate([sc_out.reshape(_NS_SC, _ROWS, _COLS), tc_out], axis=0)
    return out.reshape(sz)


# speedup vs baseline: 1.4216x; 1.4216x over previous
import functools
import jax
import jax.numpy as jnp
from jax import lax
from jax.experimental import pallas as pl
from jax.experimental.pallas import tpu as pltpu
from jax.experimental.pallas import tpu_sc as plsc

N_BINS = 20
NSEG = N_BINS - 1
_NW = 32
_TOT = 16 * 4 * 512 * 512
_PER_W = _TOT // _NW
_CHUNK = 16384
_NCHUNK = _PER_W // _CHUNK
_GRP = _CHUNK // 16


def _sc_body(x_hbm, rany_hbm, out_hbm, yrow_v, ytab_v, atab_v, btab_v,
             inb0_v, inb1_v, inb2_v, outb0_v, outb1_v, outb2_v, sem_in, sem_out):
    inb = (inb0_v, inb1_v, inb2_v)
    outb = (outb0_v, outb1_v, outb2_v)
    cid = lax.axis_index("c")
    sid = lax.axis_index("s")
    wid = sid * 2 + cid
    sample = wid // 2
    base = wid * _PER_W

    pltpu.sync_copy(rany_hbm.at[sample], yrow_v)
    v0 = yrow_v[pl.ds(0, 16)]
    v1 = yrow_v[pl.ds(16, 16)]
    lane = lax.iota(jnp.int32, 16)
    m1 = lane < (N_BINS - 16)
    big = jnp.float32(3.4e38)
    vmin = jnp.minimum(v0, jnp.where(m1, v1, big))
    vmax = jnp.maximum(v0, jnp.where(m1, v1, -big))
    for sh in (8, 4, 2, 1):
        ridx = (lane + sh) & 15
        yrow_v[pl.ds(0, 16)] = vmin
        vmin = jnp.minimum(vmin, plsc.load_gather(yrow_v, [ridx]))
        yrow_v[pl.ds(0, 16)] = vmax
        vmax = jnp.maximum(vmax, plsc.load_gather(yrow_v, [ridx]))
    ymin = vmin
    sc = 1.0 / (vmax - vmin + 1e-5)
    yn0 = (v0 - ymin) * sc
    yn1 = (v1 - ymin) * sc
    ytab_v[pl.ds(0, 16)] = yn0
    ytab_v[pl.ds(16, 16)] = yn1
    dy0 = ytab_v[pl.ds(1, 16)] - yn0
    dy1 = ytab_v[pl.ds(17, 16)] - ytab_v[pl.ds(16, 16)]
    k0 = lane.astype(jnp.float32)
    k1 = k0 + 16.0
    atab_v[pl.ds(0, 16)] = yn0 - k0 * dy0
    atab_v[pl.ds(16, 16)] = yn1 - k1 * dy1
    btab_v[pl.ds(0, 16)] = dy0
    btab_v[pl.ds(16, 16)] = dy1

    nineteen = jnp.float32(NSEG)

    def compute(buf_in, buf_out):
        @plsc.parallel_loop(0, _CHUNK, 16, unroll=16)
        def body(i):
            xv = buf_in[pl.ds(i, 16)]
            t = xv * nineteen
            idx = t.astype(jnp.int32)
            a = plsc.load_gather(atab_v, [idx])
            b = plsc.load_gather(btab_v, [idx])
            buf_out[pl.ds(i, 16)] = a + b * t

    _NB = 3
    cps_in = {}
    for g in range(min(_NB - 1, _NCHUNK)):
        cps_in[g] = pltpu.async_copy(
            x_hbm.at[pl.ds(base + g * _CHUNK, _CHUNK)], inb[g % _NB], sem_in)
    for g in range(_NCHUNK):
        if g + _NB - 1 < _NCHUNK:
            gg = g + _NB - 1
            cps_in[gg] = pltpu.async_copy(
                x_hbm.at[pl.ds(base + gg * _CHUNK, _CHUNK)], inb[gg % _NB], sem_in)
        cps_in[g].wait()
        if g >= _NB:
            pltpu.make_async_copy(
                outb[g % _NB],
                out_hbm.at[pl.ds(base + (g - _NB) * _CHUNK, _CHUNK)],
                sem_out).wait()
        compute(inb[g % _NB], outb[g % _NB])
        pltpu.async_copy(
            outb[g % _NB], out_hbm.at[pl.ds(base + g * _CHUNK, _CHUNK)], sem_out)
    for g in range(max(0, _NCHUNK - _NB), _NCHUNK):
        pltpu.make_async_copy(
            outb[g % _NB], out_hbm.at[pl.ds(base + g * _CHUNK, _CHUNK)],
            sem_out).wait()


def _sc_lut(x_flat, ran_y_pad):
    mesh = plsc.VectorSubcoreMesh(core_axis_name="c", subcore_axis_name="s")
    k = functools.partial(
        pl.kernel,
        out_type=jax.ShapeDtypeStruct((_TOT,), jnp.float32),
        mesh=mesh,
        compiler_params=pltpu.CompilerParams(needs_layout_passes=False),
        scratch_types=[
            pltpu.VMEM((32,), jnp.float32),
            pltpu.VMEM((48,), jnp.float32),
            pltpu.VMEM((48,), jnp.float32),
            pltpu.VMEM((48,), jnp.float32),
            pltpu.VMEM((_CHUNK,), jnp.float32),
            pltpu.VMEM((_CHUNK,), jnp.float32),
            pltpu.VMEM((_CHUNK,), jnp.float32),
            pltpu.VMEM((_CHUNK,), jnp.float32),
            pltpu.VMEM((_CHUNK,), jnp.float32),
            pltpu.VMEM((_CHUNK,), jnp.float32),
            pltpu.SemaphoreType.DMA,
            pltpu.SemaphoreType.DMA,
        ],
    )(_sc_body)
    return k(x_flat, ran_y_pad)


def kernel(x, ran_y):
    sz = x.shape
    x_flat = x.reshape(-1)
    ran_y_pad = jnp.pad(ran_y, ((0, 0), (0, 32 - N_BINS)))
    out = _sc_lut(x_flat, ran_y_pad)
    return out.reshape(sz)
